# baseline (speedup 1.0000x reference)
import functools

import jax
import jax.numpy as jnp
from jax import lax
from jax.experimental import pallas as pl
from jax.experimental.pallas import tpu as pltpu
from jax.experimental.pallas import tpu_sc as plsc

D = 128
NLANES = 16
NPARTS = D // NLANES
NW = 32
CHUNK = 128
PAD_ROWS = 4


def _lstm_call(q_star, h, c, w_ih, w_hh, bias):
    bsz = q_star.shape[0]
    dn = (((1,), (1,)), ((), ()))

    def body(q_ref, h_ref, c_ref, wih_ref, whh_ref, b_ref, hout_ref, cout_ref):
        gates = lax.dot_general(q_ref[...], wih_ref[...], dn,
                                preferred_element_type=jnp.float32)
        gates = gates + lax.dot_general(h_ref[...], whh_ref[...], dn,
                                        preferred_element_type=jnp.float32)
        gates = gates + b_ref[...]
        i_g = jax.nn.sigmoid(gates[:, :D])
        f_g = jax.nn.sigmoid(gates[:, D:2 * D])
        g_g = jnp.tanh(gates[:, 2 * D:3 * D])
        o_g = jax.nn.sigmoid(gates[:, 3 * D:])
        c_new = f_g * c_ref[...] + i_g * g_g
        hout_ref[...] = o_g * jnp.tanh(c_new)
        cout_ref[...] = c_new

    return pl.pallas_call(
        body,
        out_shape=(jax.ShapeDtypeStruct((bsz, D), jnp.float32),
                   jax.ShapeDtypeStruct((bsz, D), jnp.float32)),
    )(q_star, h, c, w_ih, w_hh, bias)


def _make_attn(bsz, nrows_total):
    seg_per_w = bsz // NW
    nrow_limit = nrows_total - CHUNK
    mesh = plsc.VectorSubcoreMesh(core_axis_name="c", subcore_axis_name="s",
                                  num_cores=2, num_subcores=16)
    buf_words = (CHUNK + PAD_ROWS) * D

    @functools.partial(
        pl.kernel,
        out_type=jax.ShapeDtypeStruct((bsz * D,), jnp.float32),
        mesh=mesh,
        compiler_params=pltpu.CompilerParams(needs_layout_passes=False),
        scratch_types=[
            pltpu.VMEM((buf_words,), jnp.float32),
            pltpu.VMEM((buf_words,), jnp.float32),
            pltpu.VMEM((48,), jnp.int32),
            pltpu.VMEM((seg_per_w * D,), jnp.float32),
            pltpu.VMEM((seg_per_w * D,), jnp.float32),
            pltpu.SemaphoreType.DMA,
            pltpu.SemaphoreType.DMA,
        ],
    )
    def attn(x_hbm, starts_hbm, q_hbm, r_hbm, buf_a, buf_b, sbuf, qbuf, rbuf,
             sem_a, sem_b):
        wid = lax.axis_index("s") * 2 + lax.axis_index("c")
        seg0 = wid * seg_per_w
        pltpu.sync_copy(starts_hbm.at[pl.ds(seg0, 48)], sbuf)
        pltpu.sync_copy(q_hbm.at[pl.ds(seg0 * D, seg_per_w * D)], qbuf)

        zero_v = jnp.zeros((NLANES,), jnp.float32)
        row_start = sbuf[pl.ds(0, NLANES)][0]
        row_end = sbuf[pl.ds(seg_per_w, NLANES)][0]

        def zr(i, _):
            rbuf[pl.ds(i * NLANES, NLANES)] = zero_v
            return 0
        lax.fori_loop(0, seg_per_w * D // NLANES, zr, 0)
        for j in range(PAD_ROWS * D // NLANES):
            buf_a[pl.ds(CHUNK * D + j * NLANES, NLANES)] = zero_v
            buf_b[pl.ds(CHUNK * D + j * NLANES, NLANES)] = zero_v

        total = row_end - row_start
        nchunks = (total + CHUNK - 1) // CHUNK

        def base_of(ci):
            want = row_start + ci * CHUNK
            return jnp.minimum(want, nrow_limit)

        def issue(ci, buf, sem):
            pltpu.async_copy(
                x_hbm.at[pl.ds(base_of(ci) * D, CHUNK * D)],
                buf.at[pl.ds(0, CHUNK * D)], sem)

        def wait(ci, buf, sem):
            pltpu.make_async_copy(
                x_hbm.at[pl.ds(base_of(ci) * D, CHUNK * D)],
                buf.at[pl.ds(0, CHUNK * D)], sem).wait()

        def process_chunk(ci, buf, state):
            base = base_of(ci)
            chunk_hi = jnp.minimum(row_start + (ci + 1) * CHUNK, row_end)

            def cond(st):
                return st[0] < chunk_hi

            def body(st):
                ptr, b_loc, sacc, v, qv = st
                sv = sbuf[pl.ds(b_loc, NLANES)]
                eb = sv[1]
                span_end = jnp.minimum(eb, chunk_hi)
                ngrp = (span_end - ptr + 3) // 4

                def grp(g, c):
                    sacc, v = c
                    i0 = ptr + g * 4
                    for k in range(4):
                        i = i0 + k
                        off = (i - base) * D
                        xv = [buf[pl.ds(off + NLANES * j, NLANES)]
                              for j in range(NPARTS)]
                        p = [xv[j] * qv[j] for j in range(NPARTS)]
                        acc = ((p[0] + p[1]) + (p[2] + p[3])) + \
                              ((p[4] + p[5]) + (p[6] + p[7]))
                        e = jnp.sum(acc)
                        w = jnp.exp(jnp.broadcast_to(e, (NLANES,)))
                        valid = jnp.broadcast_to(i < span_end, (NLANES,))
                        w = jnp.where(valid, w, 0.0)
                        sacc = sacc + w
                        v = tuple(v[j] + w * xv[j] for j in range(NPARTS))
                    return sacc, v

                sacc, v = lax.fori_loop(0, ngrp, grp, (sacc, v))
                done = eb <= chunk_hi
                denom = sacc + 1e-16

                @pl.when(done)
                def _():
                    for j in range(NPARTS):
                        rbuf[pl.ds(b_loc * D + NLANES * j, NLANES)] = \
                            v[j] / denom

                b_next = jnp.minimum(b_loc + 1, seg_per_w - 1)
                qv_next = tuple(qbuf[pl.ds(b_next * D + NLANES * j, NLANES)]
                                for j in range(NPARTS))
                dv = jnp.broadcast_to(done, (NLANES,))
                sacc = jnp.where(dv, 0.0, sacc)
                v = tuple(jnp.where(dv, 0.0, v[j]) for j in range(NPARTS))
                qv = tuple(jnp.where(dv, qv_next[j], qv[j])
                           for j in range(NPARTS))
                b_loc = jnp.where(done, b_loc + 1, b_loc)
                return (span_end, b_loc, sacc, v, qv)

            return lax.while_loop(cond, body, state)

        qv0 = tuple(qbuf[pl.ds(NLANES * j, NLANES)] for j in range(NPARTS))
        state = (row_start, jnp.int32(0), zero_v, (zero_v,) * NPARTS, qv0)

        @pl.when(nchunks < 0)
        def _():
            issue(0, buf_a, sem_a)

        def pair_body(p, st):
            ci0 = 2 * p
            wait(ci0, buf_a, sem_a)

            @pl.when(ci0 + 1 < nchunks)
            def _():
                issue(ci0 + 1, buf_b, sem_b)

            st = process_chunk(ci0, buf_a, st)

            def odd(st):
                wait(ci0 + 1, buf_b, sem_b)

                @pl.when(ci0 + 2 < nchunks)
                def _():
                    issue(ci0 + 2, buf_a, sem_a)

                return process_chunk(ci0 + 1, buf_b, st)

            return lax.cond(ci0 + 1 < nchunks, odd, lambda s: s, st)

        npairs = (nchunks + 1) // 2
        lax.fori_loop(0, jnp.minimum(npairs, 0), pair_body, state)
        pltpu.sync_copy(rbuf, r_hbm.at[pl.ds(seg0 * D, seg_per_w * D)])

    return attn


def kernel(x, bank_kg, batch, q_star, W_ih, W_hh, b_ih, b_hh):
    n, d = x.shape
    assert d == D
    bsz = q_star.shape[0]

    starts = jnp.searchsorted(batch, jnp.arange(bsz + 1, dtype=jnp.int32),
                              side="left", method="sort").astype(jnp.int32)
    spad = ((bsz + 48 + 7) // 8) * 8
    starts_pad = jnp.pad(starts, (0, spad - (bsz + 1)), constant_values=n)

    x_flat = x.reshape(-1)
    bias = (b_ih + b_hh).reshape(1, -1)

    attn = _make_attn(bsz, n)

    h = jnp.zeros((bsz, D), jnp.float32)
    c = jnp.zeros((bsz, D), jnp.float32)
    for _ in range(2):
        h, c = _lstm_call(q_star, h, c, W_ih, W_hh, bias)
        r_flat = (h.reshape(-1) + starts_pad[0]) * x_flat[0]
        q_star = jnp.concatenate([h, r_flat.reshape(bsz, D)], axis=1)
    return q_star


# speedup vs baseline: 4.6488x; 4.6488x over previous
import functools

import jax
import jax.numpy as jnp
from jax import lax
from jax.experimental import pallas as pl
from jax.experimental.pallas import tpu as pltpu
from jax.experimental.pallas import tpu_sc as plsc

D = 128
NLANES = 16
NPARTS = D // NLANES
NW = 32
CHUNK = 128
PAD_ROWS = 4


def _lstm_call(q_star, h, c, w_ih, w_hh, bias):
    bsz = q_star.shape[0]
    dn = (((1,), (1,)), ((), ()))

    def body(q_ref, h_ref, c_ref, wih_ref, whh_ref, b_ref, hout_ref, cout_ref):
        gates = lax.dot_general(q_ref[...], wih_ref[...], dn,
                                preferred_element_type=jnp.float32)
        gates = gates + lax.dot_general(h_ref[...], whh_ref[...], dn,
                                        preferred_element_type=jnp.float32)
        gates = gates + b_ref[...]
        i_g = jax.nn.sigmoid(gates[:, :D])
        f_g = jax.nn.sigmoid(gates[:, D:2 * D])
        g_g = jnp.tanh(gates[:, 2 * D:3 * D])
        o_g = jax.nn.sigmoid(gates[:, 3 * D:])
        c_new = f_g * c_ref[...] + i_g * g_g
        hout_ref[...] = o_g * jnp.tanh(c_new)
        cout_ref[...] = c_new

    return pl.pallas_call(
        body,
        out_shape=(jax.ShapeDtypeStruct((bsz, D), jnp.float32),
                   jax.ShapeDtypeStruct((bsz, D), jnp.float32)),
    )(q_star, h, c, w_ih, w_hh, bias)


def _make_attn(bsz, nrows_total, compute_starts):
    seg_per_w = bsz // NW
    nrow_limit = nrows_total - CHUNK
    mesh = plsc.VectorSubcoreMesh(core_axis_name="c", subcore_axis_name="s",
                                  num_cores=2, num_subcores=16)
    buf_words = (CHUNK + PAD_ROWS) * D
    spad = bsz + 48

    r_type = jax.ShapeDtypeStruct((bsz * D,), jnp.float32)
    if compute_starts:
        out_type = (r_type, jax.ShapeDtypeStruct((spad,), jnp.int32))
    else:
        out_type = r_type

    @functools.partial(
        pl.kernel,
        out_type=out_type,
        mesh=mesh,
        compiler_params=pltpu.CompilerParams(needs_layout_passes=False),
        scratch_types=[
            pltpu.VMEM((buf_words,), jnp.float32),
            pltpu.VMEM((buf_words,), jnp.float32),
            pltpu.VMEM((48,), jnp.int32),
            pltpu.VMEM((seg_per_w * D,), jnp.float32),
            pltpu.VMEM((seg_per_w * D,), jnp.float32),
            pltpu.VMEM((48,), jnp.int32),
            pltpu.VMEM((48,), jnp.int32),
            pltpu.SemaphoreType.DMA,
            pltpu.SemaphoreType.DMA,
        ],
    )
    def attn(x_hbm, starts_hbm, q_hbm, *rest):
        if compute_starts:
            r_hbm, sout_hbm, buf_a, buf_b, sbuf, qbuf, rbuf, ibuf, pbuf, \
                sem_a, sem_b = rest
        else:
            r_hbm, buf_a, buf_b, sbuf, qbuf, rbuf, ibuf, pbuf, \
                sem_a, sem_b = rest
        wid = lax.axis_index("s") * 2 + lax.axis_index("c")
        seg0 = wid * seg_per_w
        if not compute_starts:
            pltpu.sync_copy(starts_hbm.at[pl.ds(seg0, 48)], sbuf)
        else:
            lanes = lax.broadcasted_iota(jnp.int32, (NLANES,), 0)
            tgt = [seg0 + 16 * k + lanes for k in range(3)]
            lo = [jnp.zeros((NLANES,), jnp.int32) for _ in range(3)]
            hi = [jnp.full((NLANES,), nrows_total, jnp.int32) for _ in range(3)]
            for _ in range(17):
                mids = []
                for k in range(3):
                    mid = (lo[k] + hi[k]) >> 1
                    mids.append(mid)
                    ibuf[pl.ds(16 * k, NLANES)] = \
                        jnp.minimum(mid, nrows_total - 1)
                pltpu.async_copy(starts_hbm.at[ibuf], pbuf, sem_a)
                pltpu.make_async_copy(starts_hbm.at[ibuf], pbuf, sem_a).wait()
                for k in range(3):
                    probe = pbuf[pl.ds(16 * k, NLANES)]
                    open_ = lo[k] < hi[k]
                    less = probe < tgt[k]
                    lo[k] = jnp.where(open_ & less, mids[k] + 1, lo[k])
                    hi[k] = jnp.where(open_ & jnp.logical_not(less),
                                      mids[k], hi[k])
            for k in range(3):
                sbuf[pl.ds(16 * k, NLANES)] = lo[k]
            pltpu.sync_copy(sbuf.at[pl.ds(0, seg_per_w)],
                            sout_hbm.at[pl.ds(seg0, seg_per_w)])

            @pl.when(wid == NW - 1)
            def _():
                pltpu.sync_copy(sbuf.at[pl.ds(seg_per_w, 16)],
                                sout_hbm.at[pl.ds(seg0 + seg_per_w, 16)])
        pltpu.sync_copy(q_hbm.at[pl.ds(seg0 * D, seg_per_w * D)], qbuf)

        zero_v = jnp.zeros((NLANES,), jnp.float32)
        row_start = sbuf[pl.ds(0, NLANES)][0]
        row_end = sbuf[pl.ds(seg_per_w, NLANES)][0]

        def zr(i, _):
            rbuf[pl.ds(i * NLANES, NLANES)] = zero_v
            return 0
        lax.fori_loop(0, seg_per_w * D // NLANES, zr, 0)
        for j in range(PAD_ROWS * D // NLANES):
            buf_a[pl.ds(CHUNK * D + j * NLANES, NLANES)] = zero_v
            buf_b[pl.ds(CHUNK * D + j * NLANES, NLANES)] = zero_v

        total = row_end - row_start
        nchunks = (total + CHUNK - 1) // CHUNK

        def base_of(ci):
            want = row_start + ci * CHUNK
            return jnp.minimum(want, nrow_limit)

        def issue(ci, buf, sem):
            pltpu.async_copy(
                x_hbm.at[pl.ds(base_of(ci) * D, CHUNK * D)],
                buf.at[pl.ds(0, CHUNK * D)], sem)

        def wait(ci, buf, sem):
            pltpu.make_async_copy(
                x_hbm.at[pl.ds(base_of(ci) * D, CHUNK * D)],
                buf.at[pl.ds(0, CHUNK * D)], sem).wait()

        def process_chunk(ci, buf, state):
            base = base_of(ci)
            chunk_hi = jnp.minimum(row_start + (ci + 1) * CHUNK, row_end)

            def cond(st):
                return st[0] < chunk_hi

            def body(st):
                ptr, b_loc, sacc, v, qv = st
                sv = sbuf[pl.ds(b_loc, NLANES)]
                eb = sv[1]
                span_end = jnp.minimum(eb, chunk_hi)
                ngrp = (span_end - ptr + 3) // 4

                def grp(g, c):
                    sacc, v = c
                    i0 = ptr + g * 4
                    for k in range(4):
                        i = i0 + k
                        off = (i - base) * D
                        xv = [buf[pl.ds(off + NLANES * j, NLANES)]
                              for j in range(NPARTS)]
                        p = [xv[j] * qv[j] for j in range(NPARTS)]
                        acc = ((p[0] + p[1]) + (p[2] + p[3])) + \
                              ((p[4] + p[5]) + (p[6] + p[7]))
                        e = jnp.sum(acc)
                        w = jnp.exp(jnp.broadcast_to(e, (NLANES,)))
                        valid = jnp.broadcast_to(i < span_end, (NLANES,))
                        w = jnp.where(valid, w, 0.0)
                        sacc = sacc + w
                        v = tuple(v[j] + w * xv[j] for j in range(NPARTS))
                    return sacc, v

                sacc, v = lax.fori_loop(0, ngrp, grp, (sacc, v))
                done = eb <= chunk_hi
                denom = sacc + 1e-16

                @pl.when(done)
                def _():
                    for j in range(NPARTS):
                        rbuf[pl.ds(b_loc * D + NLANES * j, NLANES)] = \
                            v[j] / denom

                b_next = jnp.minimum(b_loc + 1, seg_per_w - 1)
                qv_next = tuple(qbuf[pl.ds(b_next * D + NLANES * j, NLANES)]
                                for j in range(NPARTS))
                dv = jnp.broadcast_to(done, (NLANES,))
                sacc = jnp.where(dv, 0.0, sacc)
                v = tuple(jnp.where(dv, 0.0, v[j]) for j in range(NPARTS))
                qv = tuple(jnp.where(dv, qv_next[j], qv[j])
                           for j in range(NPARTS))
                b_loc = jnp.where(done, b_loc + 1, b_loc)
                return (span_end, b_loc, sacc, v, qv)

            return lax.while_loop(cond, body, state)

        qv0 = tuple(qbuf[pl.ds(NLANES * j, NLANES)] for j in range(NPARTS))
        state = (row_start, jnp.int32(0), zero_v, (zero_v,) * NPARTS, qv0)

        @pl.when(nchunks > 0)
        def _():
            issue(0, buf_a, sem_a)

        def pair_body(p, st):
            ci0 = 2 * p
            wait(ci0, buf_a, sem_a)

            @pl.when(ci0 + 1 < nchunks)
            def _():
                issue(ci0 + 1, buf_b, sem_b)

            st = process_chunk(ci0, buf_a, st)

            def odd(st):
                wait(ci0 + 1, buf_b, sem_b)

                @pl.when(ci0 + 2 < nchunks)
                def _():
                    issue(ci0 + 2, buf_a, sem_a)

                return process_chunk(ci0 + 1, buf_b, st)

            return lax.cond(ci0 + 1 < nchunks, odd, lambda s: s, st)

        npairs = (nchunks + 1) // 2
        lax.fori_loop(0, npairs, pair_body, state)
        pltpu.sync_copy(rbuf, r_hbm.at[pl.ds(seg0 * D, seg_per_w * D)])

    return attn


def kernel(x, bank_kg, batch, q_star, W_ih, W_hh, b_ih, b_hh):
    n, d = x.shape
    assert d == D
    bsz = q_star.shape[0]

    x_flat = x.reshape(-1)
    bias = (b_ih + b_hh).reshape(1, -1)

    attn1 = _make_attn(bsz, n, True)
    attn2 = _make_attn(bsz, n, False)

    h = jnp.zeros((bsz, D), jnp.float32)
    c = jnp.zeros((bsz, D), jnp.float32)

    h, c = _lstm_call(q_star, h, c, W_ih, W_hh, bias)
    r_flat, starts_arr = attn1(x_flat, batch, h.reshape(-1))
    q_star = jnp.concatenate([h, r_flat.reshape(bsz, D)], axis=1)

    h, c = _lstm_call(q_star, h, c, W_ih, W_hh, bias)
    r_flat = attn2(x_flat, starts_arr, h.reshape(-1))
    return jnp.concatenate([h, r_flat.reshape(bsz, D)], axis=1)


# speedup vs baseline: 4.7073x; 1.0126x over previous
import functools

import jax
import jax.numpy as jnp
from jax import lax
from jax.experimental import pallas as pl
from jax.experimental.pallas import tpu as pltpu
from jax.experimental.pallas import tpu_sc as plsc

D = 128
NLANES = 16
NPARTS = D // NLANES
NW = 32
CHUNK = 256
PAD_ROWS = 4


def _lstm_call(q_star, h, c, w_ih, w_hh, bias):
    bsz = q_star.shape[0]
    dn = (((1,), (1,)), ((), ()))

    def body(q_ref, h_ref, c_ref, wih_ref, whh_ref, b_ref, hout_ref, cout_ref):
        gates = lax.dot_general(q_ref[...], wih_ref[...], dn,
                                preferred_element_type=jnp.float32)
        gates = gates + lax.dot_general(h_ref[...], whh_ref[...], dn,
                                        preferred_element_type=jnp.float32)
        gates = gates + b_ref[...]
        i_g = jax.nn.sigmoid(gates[:, :D])
        f_g = jax.nn.sigmoid(gates[:, D:2 * D])
        g_g = jnp.tanh(gates[:, 2 * D:3 * D])
        o_g = jax.nn.sigmoid(gates[:, 3 * D:])
        c_new = f_g * c_ref[...] + i_g * g_g
        hout_ref[...] = o_g * jnp.tanh(c_new)
        cout_ref[...] = c_new

    return pl.pallas_call(
        body,
        out_shape=(jax.ShapeDtypeStruct((bsz, D), jnp.float32),
                   jax.ShapeDtypeStruct((bsz, D), jnp.float32)),
    )(q_star, h, c, w_ih, w_hh, bias)


def _make_attn(bsz, nrows_total, compute_starts):
    seg_per_w = bsz // NW
    nrow_limit = nrows_total - CHUNK
    mesh = plsc.VectorSubcoreMesh(core_axis_name="c", subcore_axis_name="s",
                                  num_cores=2, num_subcores=16)
    buf_words = (CHUNK + PAD_ROWS) * D
    spad = bsz + 48

    r_type = jax.ShapeDtypeStruct((bsz * D,), jnp.float32)
    if compute_starts:
        out_type = (r_type, jax.ShapeDtypeStruct((spad,), jnp.int32))
    else:
        out_type = r_type

    @functools.partial(
        pl.kernel,
        out_type=out_type,
        mesh=mesh,
        compiler_params=pltpu.CompilerParams(needs_layout_passes=False),
        scratch_types=[
            pltpu.VMEM((buf_words,), jnp.float32),
            pltpu.VMEM((buf_words,), jnp.float32),
            pltpu.VMEM((48,), jnp.int32),
            pltpu.VMEM((seg_per_w * D,), jnp.float32),
            pltpu.VMEM((seg_per_w * D,), jnp.float32),
            pltpu.VMEM((48,), jnp.int32),
            pltpu.VMEM((48,), jnp.int32),
            pltpu.SemaphoreType.DMA,
            pltpu.SemaphoreType.DMA,
        ],
    )
    def attn(x_hbm, starts_hbm, q_hbm, *rest):
        if compute_starts:
            r_hbm, sout_hbm, buf_a, buf_b, sbuf, qbuf, rbuf, ibuf, pbuf, \
                sem_a, sem_b = rest
        else:
            r_hbm, buf_a, buf_b, sbuf, qbuf, rbuf, ibuf, pbuf, \
                sem_a, sem_b = rest
        wid = lax.axis_index("s") * 2 + lax.axis_index("c")
        seg0 = wid * seg_per_w
        if not compute_starts:
            pltpu.sync_copy(starts_hbm.at[pl.ds(seg0, 48)], sbuf)
        else:
            lanes = lax.broadcasted_iota(jnp.int32, (NLANES,), 0)
            tgt = [seg0 + 16 * k + lanes for k in range(3)]
            lo = [jnp.zeros((NLANES,), jnp.int32) for _ in range(3)]
            hi = [jnp.full((NLANES,), nrows_total, jnp.int32) for _ in range(3)]
            for _ in range(17):
                mids = []
                for k in range(3):
                    mid = (lo[k] + hi[k]) >> 1
                    mids.append(mid)
                    ibuf[pl.ds(16 * k, NLANES)] = \
                        jnp.minimum(mid, nrows_total - 1)
                pltpu.async_copy(starts_hbm.at[ibuf], pbuf, sem_a)
                pltpu.make_async_copy(starts_hbm.at[ibuf], pbuf, sem_a).wait()
                for k in range(3):
                    probe = pbuf[pl.ds(16 * k, NLANES)]
                    open_ = lo[k] < hi[k]
                    less = probe < tgt[k]
                    lo[k] = jnp.where(open_ & less, mids[k] + 1, lo[k])
                    hi[k] = jnp.where(open_ & jnp.logical_not(less),
                                      mids[k], hi[k])
            for k in range(3):
                sbuf[pl.ds(16 * k, NLANES)] = lo[k]
            pltpu.sync_copy(sbuf.at[pl.ds(0, seg_per_w)],
                            sout_hbm.at[pl.ds(seg0, seg_per_w)])

            @pl.when(wid == NW - 1)
            def _():
                pltpu.sync_copy(sbuf.at[pl.ds(seg_per_w, 16)],
                                sout_hbm.at[pl.ds(seg0 + seg_per_w, 16)])
        pltpu.sync_copy(q_hbm.at[pl.ds(seg0 * D, seg_per_w * D)], qbuf)

        zero_v = jnp.zeros((NLANES,), jnp.float32)
        row_start = sbuf[pl.ds(0, NLANES)][0]
        row_end = sbuf[pl.ds(seg_per_w, NLANES)][0]

        def zr(i, _):
            rbuf[pl.ds(i * NLANES, NLANES)] = zero_v
            return 0
        lax.fori_loop(0, seg_per_w * D // NLANES, zr, 0)
        for j in range(PAD_ROWS * D // NLANES):
            buf_a[pl.ds(CHUNK * D + j * NLANES, NLANES)] = zero_v
            buf_b[pl.ds(CHUNK * D + j * NLANES, NLANES)] = zero_v

        total = row_end - row_start
        nchunks = (total + CHUNK - 1) // CHUNK

        def base_of(ci):
            want = row_start + ci * CHUNK
            return jnp.minimum(want, nrow_limit)

        def issue(ci, buf, sem):
            pltpu.async_copy(
                x_hbm.at[pl.ds(base_of(ci) * D, CHUNK * D)],
                buf.at[pl.ds(0, CHUNK * D)], sem)

        def wait(ci, buf, sem):
            pltpu.make_async_copy(
                x_hbm.at[pl.ds(base_of(ci) * D, CHUNK * D)],
                buf.at[pl.ds(0, CHUNK * D)], sem).wait()

        def process_chunk(ci, buf, state):
            base = base_of(ci)
            chunk_hi = jnp.minimum(row_start + (ci + 1) * CHUNK, row_end)

            def cond(st):
                return st[0] < chunk_hi

            def body(st):
                ptr, b_loc, sacc, v, qv = st
                sv = sbuf[pl.ds(b_loc, NLANES)]
                eb = sv[1]
                span_end = jnp.minimum(eb, chunk_hi)
                ngrp = (span_end - ptr + 3) // 4

                def grp(g, c):
                    sacc, v = c
                    i0 = ptr + g * 4
                    for k in range(4):
                        i = i0 + k
                        off = (i - base) * D
                        xv = [buf[pl.ds(off + NLANES * j, NLANES)]
                              for j in range(NPARTS)]
                        p = [xv[j] * qv[j] for j in range(NPARTS)]
                        acc = ((p[0] + p[1]) + (p[2] + p[3])) + \
                              ((p[4] + p[5]) + (p[6] + p[7]))
                        e = jnp.sum(acc)
                        w = jnp.exp(jnp.broadcast_to(e, (NLANES,)))
                        valid = jnp.broadcast_to(i < span_end, (NLANES,))
                        w = jnp.where(valid, w, 0.0)
                        sacc = sacc + w
                        v = tuple(v[j] + w * xv[j] for j in range(NPARTS))
                    return sacc, v

                sacc, v = lax.fori_loop(0, ngrp, grp, (sacc, v))
                done = eb <= chunk_hi
                denom = sacc + 1e-16

                @pl.when(done)
                def _():
                    for j in range(NPARTS):
                        rbuf[pl.ds(b_loc * D + NLANES * j, NLANES)] = \
                            v[j] / denom

                b_next = jnp.minimum(b_loc + 1, seg_per_w - 1)
                qv_next = tuple(qbuf[pl.ds(b_next * D + NLANES * j, NLANES)]
                                for j in range(NPARTS))
                dv = jnp.broadcast_to(done, (NLANES,))
                sacc = jnp.where(dv, 0.0, sacc)
                v = tuple(jnp.where(dv, 0.0, v[j]) for j in range(NPARTS))
                qv = tuple(jnp.where(dv, qv_next[j], qv[j])
                           for j in range(NPARTS))
                b_loc = jnp.where(done, b_loc + 1, b_loc)
                return (span_end, b_loc, sacc, v, qv)

            return lax.while_loop(cond, body, state)

        qv0 = tuple(qbuf[pl.ds(NLANES * j, NLANES)] for j in range(NPARTS))
        state = (row_start, jnp.int32(0), zero_v, (zero_v,) * NPARTS, qv0)

        @pl.when(nchunks > 0)
        def _():
            issue(0, buf_a, sem_a)

        def pair_body(p, st):
            ci0 = 2 * p
            wait(ci0, buf_a, sem_a)

            @pl.when(ci0 + 1 < nchunks)
            def _():
                issue(ci0 + 1, buf_b, sem_b)

            st = process_chunk(ci0, buf_a, st)

            def odd(st):
                wait(ci0 + 1, buf_b, sem_b)

                @pl.when(ci0 + 2 < nchunks)
                def _():
                    issue(ci0 + 2, buf_a, sem_a)

                return process_chunk(ci0 + 1, buf_b, st)

            return lax.cond(ci0 + 1 < nchunks, odd, lambda s: s, st)

        npairs = (nchunks + 1) // 2
        lax.fori_loop(0, npairs, pair_body, state)
        pltpu.sync_copy(rbuf, r_hbm.at[pl.ds(seg0 * D, seg_per_w * D)])

    return attn


def kernel(x, bank_kg, batch, q_star, W_ih, W_hh, b_ih, b_hh):
    n, d = x.shape
    assert d == D
    bsz = q_star.shape[0]

    x_flat = x.reshape(-1)
    bias = (b_ih + b_hh).reshape(1, -1)

    attn1 = _make_attn(bsz, n, True)
    attn2 = _make_attn(bsz, n, False)

    h = jnp.zeros((bsz, D), jnp.float32)
    c = jnp.zeros((bsz, D), jnp.float32)

    h, c = _lstm_call(q_star, h, c, W_ih, W_hh, bias)
    r_flat, starts_arr = attn1(x_flat, batch, h.reshape(-1))
    q_star = jnp.concatenate([h, r_flat.reshape(bsz, D)], axis=1)

    h, c = _lstm_call(q_star, h, c, W_ih, W_hh, bias)
    r_flat = attn2(x_flat, starts_arr, h.reshape(-1))
    return jnp.concatenate([h, r_flat.reshape(bsz, D)], axis=1)
